# Initial kernel scaffold; baseline (speedup 1.0000x reference)
import jax
import jax.numpy as jnp
from jax import lax
from jax.experimental import pallas as pl
from jax.experimental.pallas import tpu as pltpu
from jax.experimental.pallas import tpu_sc as plsc

_C = 3
_F = 5
_K = _F * _F
_HO = 48
_WO = 48
_HI = _HO + _F - 1
_WI = _WO + _F - 1
_NPIX = _HO * _WO
_NWORKERS = 32
_PPW = _NPIX // _NWORKERS
_NVEC = (_PPW + 15) // 16


def _dsep_body(inp_hbm, vt_hbm, ht_hbm, ox_hbm, oy_hbm, mk_hbm, out_hbm,
               inp_v, vt_v, ht_v, ox_v, oy_v, mk_v, out_v):
  wid = lax.axis_index("s") * 2 + lax.axis_index("c")
  base = wid * _PPW

  pltpu.sync_copy(inp_hbm, inp_v)
  pltpu.sync_copy(vt_hbm.at[pl.ds(base * _F, _PPW * _F)], vt_v)
  pltpu.sync_copy(ht_hbm.at[pl.ds(base * _F, _PPW * _F)], ht_v)
  pltpu.sync_copy(ox_hbm.at[pl.ds(base * _K, _PPW * _K)], ox_v)
  pltpu.sync_copy(oy_hbm.at[pl.ds(base * _K, _PPW * _K)], oy_v)
  pltpu.sync_copy(mk_hbm.at[pl.ds(base * _K, _PPW * _K)], mk_v)

  iota = lax.broadcasted_iota(jnp.int32, (16,), 0)
  zero = jnp.zeros((16,), jnp.float32)

  for vec in range(_NVEC):
    lp = iota + (vec * 16)
    if (vec + 1) * 16 > _PPW:
      lp = jnp.minimum(lp, _PPW - 1)
    pix = lp + base
    h = pix // _WO
    w = pix - h * _WO
    h_f = h.astype(jnp.float32)
    w_f = w.astype(jnp.float32)
    idx_k = lp * _K
    idx_f = lp * _F

    def body(k, accs, idx_k=idx_k, idx_f=idx_f, h_f=h_f, w_f=w_f):
      a0, a1, a2 = accs
      fy = k // _F
      fx = k - fy * _F
      ox = plsc.load_gather(ox_v, [idx_k + k])
      oy = plsc.load_gather(oy_v, [idx_k + k])
      mk = plsc.load_gather(mk_v, [idx_k + k])
      vv = plsc.load_gather(vt_v, [idx_f + fy])
      hh = plsc.load_gather(ht_v, [idx_f + fx])
      pos_x = oy + (w_f + (fx.astype(jnp.float32) - 1.0))
      pos_y = ox + (h_f + (fy.astype(jnp.float32) - 1.0))
      pos_x = jnp.clip(pos_x, 0.0, float(_WI - 1))
      pos_y = jnp.clip(pos_y, 0.0, float(_HI - 1))
      left = pos_x.astype(jnp.int32)
      top = pos_y.astype(jnp.int32)
      right = jnp.minimum(left + 1, _WI - 1)
      bot = jnp.minimum(top + 1, _HI - 1)
      wx = 1.0 - (pos_x - left.astype(jnp.float32))
      wy = 1.0 - (pos_y - top.astype(jnp.float32))
      row_t = top * _WI
      row_b = bot * _WI
      i_tl = row_t + left
      i_tr = row_t + right
      i_bl = row_b + left
      i_br = row_b + right
      wgt = vv * hh * mk
      outs = []
      for c, acc in zip(range(_C), (a0, a1, a2)):
        off = c * (_HI * _WI)
        tl = plsc.load_gather(inp_v, [i_tl + off])
        tr = plsc.load_gather(inp_v, [i_tr + off])
        bl = plsc.load_gather(inp_v, [i_bl + off])
        br = plsc.load_gather(inp_v, [i_br + off])
        top_l = tr + wx * (tl - tr)
        bot_l = br + wx * (bl - br)
        val = bot_l + wy * (top_l - bot_l)
        outs.append(acc + val * wgt)
      return tuple(outs)

    a0, a1, a2 = lax.fori_loop(0, _K, body, (zero, zero, zero))
    out_v[pl.ds(0 * 80 + vec * 16, 16)] = a0
    out_v[pl.ds(1 * 80 + vec * 16, 16)] = a1
    out_v[pl.ds(2 * 80 + vec * 16, 16)] = a2

  for c in range(_C):
    pltpu.sync_copy(out_v.at[pl.ds(c * 80, _PPW)],
                    out_hbm.at[pl.ds(c * _NPIX + base, _PPW)])


@jax.jit
def _dsepconv_sc(inp_flat, vt, ht, ox, oy, mk):
  mesh = plsc.VectorSubcoreMesh(core_axis_name="c", subcore_axis_name="s")
  run = pl.kernel(
      _dsep_body,
      out_type=jax.ShapeDtypeStruct((_C * _NPIX,), jnp.float32),
      mesh=mesh,
      scratch_types=[
          pltpu.VMEM((_C * _HI * _WI,), jnp.float32),
          pltpu.VMEM((_PPW * _F,), jnp.float32),
          pltpu.VMEM((_PPW * _F,), jnp.float32),
          pltpu.VMEM((_PPW * _K,), jnp.float32),
          pltpu.VMEM((_PPW * _K,), jnp.float32),
          pltpu.VMEM((_PPW * _K,), jnp.float32),
          pltpu.VMEM((_C * 80,), jnp.float32),
      ],
  )
  return run(inp_flat, vt, ht, ox, oy, mk)


def kernel(tensorInput, tensorVertical, tensorHorizontal, tensorOffsetX,
           tensorOffsetY, tensorMask):
  inp_flat = tensorInput.reshape(_C * _HI * _WI)
  vt = tensorVertical.reshape(_F, _NPIX).T.reshape(-1)
  ht = tensorHorizontal.reshape(_F, _NPIX).T.reshape(-1)
  ox = tensorOffsetX.reshape(_K, _NPIX).T.reshape(-1)
  oy = tensorOffsetY.reshape(_K, _NPIX).T.reshape(-1)
  mk = tensorMask.reshape(_K, _NPIX).T.reshape(-1)
  out = _dsepconv_sc(inp_flat, vt, ht, ox, oy, mk)
  return out.reshape(1, _C, _HO, _WO)


# speedup vs baseline: 141.2476x; 141.2476x over previous
import jax
import jax.numpy as jnp
from jax import lax
from jax.experimental import pallas as pl
from jax.experimental.pallas import tpu as pltpu
from jax.experimental.pallas import tpu_sc as plsc

_C = 3
_F = 5
_K = _F * _F
_HO = 48
_WO = 48
_HI = _HO + _F - 1
_WI = _WO + _F - 1
_NPIX = _HO * _WO
_NWORKERS = 32
_PPW = _NPIX // _NWORKERS
_NVEC = (_PPW + 15) // 16


def _dsep_body(inp_hbm, vt_hbm, ht_hbm, ox_hbm, oy_hbm, mk_hbm, out_hbm,
               inp_v, vt_v, ht_v, ox_v, oy_v, mk_v, out_v):
  wid = lax.axis_index("s") * 2 + lax.axis_index("c")
  base = wid * _PPW

  pltpu.sync_copy(inp_hbm, inp_v)
  pltpu.sync_copy(vt_hbm.at[pl.ds(base * _F, _PPW * _F)], vt_v)
  pltpu.sync_copy(ht_hbm.at[pl.ds(base * _F, _PPW * _F)], ht_v)
  pltpu.sync_copy(ox_hbm.at[pl.ds(base * _K, _PPW * _K)], ox_v)
  pltpu.sync_copy(oy_hbm.at[pl.ds(base * _K, _PPW * _K)], oy_v)
  pltpu.sync_copy(mk_hbm.at[pl.ds(base * _K, _PPW * _K)], mk_v)

  iota = lax.broadcasted_iota(jnp.int32, (16,), 0)
  zero = jnp.zeros((16,), jnp.float32)

  for vec in range(_NVEC):
    lp = iota + (vec * 16)
    if (vec + 1) * 16 > _PPW:
      lp = jnp.minimum(lp, _PPW - 1)
    pix = lp + base
    h = lax.div(pix, _WO)
    w = pix - h * _WO
    h_f = h.astype(jnp.float32)
    w_f = w.astype(jnp.float32)
    idx_k = lp * _K
    idx_f = lp * _F

    def body(k, accs, idx_k=idx_k, idx_f=idx_f, h_f=h_f, w_f=w_f):
      a0, a1, a2 = accs
      fy = lax.div(k, _F)
      fx = k - fy * _F
      ox = plsc.load_gather(ox_v, [idx_k + k])
      oy = plsc.load_gather(oy_v, [idx_k + k])
      mk = plsc.load_gather(mk_v, [idx_k + k])
      vv = plsc.load_gather(vt_v, [idx_f + fy])
      hh = plsc.load_gather(ht_v, [idx_f + fx])
      pos_x = oy + (w_f + (fx.astype(jnp.float32) - 1.0))
      pos_y = ox + (h_f + (fy.astype(jnp.float32) - 1.0))
      pos_x = jnp.minimum(jnp.maximum(pos_x, 0.0), float(_WI - 1))
      pos_y = jnp.minimum(jnp.maximum(pos_y, 0.0), float(_HI - 1))
      left = pos_x.astype(jnp.int32)
      top = pos_y.astype(jnp.int32)
      right = jnp.minimum(left + 1, _WI - 1)
      bot = jnp.minimum(top + 1, _HI - 1)
      wx = 1.0 - (pos_x - left.astype(jnp.float32))
      wy = 1.0 - (pos_y - top.astype(jnp.float32))
      row_t = top * _WI
      row_b = bot * _WI
      i_tl = row_t + left
      i_tr = row_t + right
      i_bl = row_b + left
      i_br = row_b + right
      wgt = vv * hh * mk
      outs = []
      for c, acc in zip(range(_C), (a0, a1, a2)):
        off = c * (_HI * _WI)
        tl = plsc.load_gather(inp_v, [i_tl + off])
        tr = plsc.load_gather(inp_v, [i_tr + off])
        bl = plsc.load_gather(inp_v, [i_bl + off])
        br = plsc.load_gather(inp_v, [i_br + off])
        top_l = tr + wx * (tl - tr)
        bot_l = br + wx * (bl - br)
        val = bot_l + wy * (top_l - bot_l)
        outs.append(acc + val * wgt)
      return tuple(outs)

    a0, a1, a2 = lax.fori_loop(0, _K, body, (zero, zero, zero))
    out_v[pl.ds(0 * 80 + vec * 16, 16)] = a0
    out_v[pl.ds(1 * 80 + vec * 16, 16)] = a1
    out_v[pl.ds(2 * 80 + vec * 16, 16)] = a2

  for c in range(_C):
    pltpu.sync_copy(out_v.at[pl.ds(c * 80, _PPW)],
                    out_hbm.at[pl.ds(c * _NPIX + base, _PPW)])


@jax.jit
def _dsepconv_sc(inp_flat, vt, ht, ox, oy, mk):
  mesh = plsc.VectorSubcoreMesh(core_axis_name="c", subcore_axis_name="s")
  run = pl.kernel(
      _dsep_body,
      out_type=jax.ShapeDtypeStruct((_C * _NPIX,), jnp.float32),
      mesh=mesh,
      compiler_params=pltpu.CompilerParams(needs_layout_passes=False),
      scratch_types=[
          pltpu.VMEM((_C * _HI * _WI,), jnp.float32),
          pltpu.VMEM((_PPW * _F,), jnp.float32),
          pltpu.VMEM((_PPW * _F,), jnp.float32),
          pltpu.VMEM((_PPW * _K,), jnp.float32),
          pltpu.VMEM((_PPW * _K,), jnp.float32),
          pltpu.VMEM((_PPW * _K,), jnp.float32),
          pltpu.VMEM((_C * 80,), jnp.float32),
      ],
  )
  return run(inp_flat, vt, ht, ox, oy, mk)


def kernel(tensorInput, tensorVertical, tensorHorizontal, tensorOffsetX,
           tensorOffsetY, tensorMask):
  inp_flat = tensorInput.reshape(_C * _HI * _WI)
  vt = tensorVertical.reshape(_F, _NPIX).T.reshape(-1)
  ht = tensorHorizontal.reshape(_F, _NPIX).T.reshape(-1)
  ox = tensorOffsetX.reshape(_K, _NPIX).T.reshape(-1)
  oy = tensorOffsetY.reshape(_K, _NPIX).T.reshape(-1)
  mk = tensorMask.reshape(_K, _NPIX).T.reshape(-1)
  out = _dsepconv_sc(inp_flat, vt, ht, ox, oy, mk)
  return out.reshape(1, _C, _HO, _WO)


# speedup vs baseline: 171.1746x; 1.2119x over previous
import jax
import jax.numpy as jnp
from jax import lax
from jax.experimental import pallas as pl
from jax.experimental.pallas import tpu as pltpu
from jax.experimental.pallas import tpu_sc as plsc

_C = 3
_F = 5
_K = _F * _F
_HO = 48
_WO = 48
_HI = _HO + _F - 1
_WI = _WO + _F - 1
_NPIX = _HO * _WO
_NWORKERS = 32
_PPW = _NPIX // _NWORKERS
_NVEC = (_PPW + 15) // 16


def _dsep_body(inp_hbm, vt_hbm, ht_hbm, ox_hbm, oy_hbm, mk_hbm, out_hbm,
               inp_v, vt_v, ht_v, ox_v, oy_v, mk_v, out_v, sem):
  wid = lax.axis_index("s") * 2 + lax.axis_index("c")
  base = wid * _PPW

  cols = pl.ds(base, _PPW)
  copies = [
      pltpu.async_copy(inp_hbm, inp_v, sem),
      pltpu.async_copy(vt_hbm.at[:, cols], vt_v, sem),
      pltpu.async_copy(ht_hbm.at[:, cols], ht_v, sem),
      pltpu.async_copy(ox_hbm.at[:, cols], ox_v, sem),
      pltpu.async_copy(oy_hbm.at[:, cols], oy_v, sem),
      pltpu.async_copy(mk_hbm.at[:, cols], mk_v, sem),
  ]
  for cp in copies:
    cp.wait()

  iota = lax.broadcasted_iota(jnp.int32, (16,), 0)
  zero = jnp.zeros((16,), jnp.float32)
  zero_i = jnp.zeros((16,), jnp.int32)

  for vec in range(_NVEC):
    lp = iota + (vec * 16)
    if (vec + 1) * 16 > _PPW:
      lp = jnp.minimum(lp, _PPW - 1)
    pix = lp + base
    h = lax.div(pix, _WO)
    w = pix - h * _WO
    h_f = h.astype(jnp.float32)
    w_f = w.astype(jnp.float32)
    hh_c = [plsc.load_gather(ht_v, [zero_i + fx, lp]) for fx in range(_F)]

    def body(fy, accs, lp=lp, h_f=h_f, w_f=w_f, hh_c=hh_c):
      a0, a1, a2 = accs
      fy_vec = zero_i + fy
      vv = plsc.load_gather(vt_v, [fy_vec, lp])
      fy_f = fy.astype(jnp.float32)
      for fx in range(_F):
        k_vec = fy_vec * _F + fx
        ox = plsc.load_gather(ox_v, [k_vec, lp])
        oy = plsc.load_gather(oy_v, [k_vec, lp])
        mk = plsc.load_gather(mk_v, [k_vec, lp])
        pos_x = oy + (w_f + float(fx - 1))
        pos_y = ox + (h_f + (fy_f - 1.0))
        pos_x = jnp.minimum(jnp.maximum(pos_x, 0.0), float(_WI - 1))
        pos_y = jnp.minimum(jnp.maximum(pos_y, 0.0), float(_HI - 1))
        left = pos_x.astype(jnp.int32)
        top = pos_y.astype(jnp.int32)
        right = jnp.minimum(left + 1, _WI - 1)
        bot = jnp.minimum(top + 1, _HI - 1)
        wx = 1.0 - (pos_x - left.astype(jnp.float32))
        wy = 1.0 - (pos_y - top.astype(jnp.float32))
        row_t = top * _WI
        row_b = bot * _WI
        i_tl = row_t + left
        i_tr = row_t + right
        i_bl = row_b + left
        i_br = row_b + right
        wgt = vv * hh_c[fx] * mk
        outs = []
        for acc, off in zip((a0, a1, a2), (0, _HI * _WI, 2 * _HI * _WI)):
          tl = plsc.load_gather(inp_v, [i_tl + off])
          tr = plsc.load_gather(inp_v, [i_tr + off])
          bl = plsc.load_gather(inp_v, [i_bl + off])
          br = plsc.load_gather(inp_v, [i_br + off])
          top_l = tr + wx * (tl - tr)
          bot_l = br + wx * (bl - br)
          val = bot_l + wy * (top_l - bot_l)
          outs.append(acc + val * wgt)
        a0, a1, a2 = outs
      return a0, a1, a2

    a0, a1, a2 = lax.fori_loop(0, _F, body, (zero, zero, zero))
    out_v[pl.ds(0 * 80 + vec * 16, 16)] = a0
    out_v[pl.ds(1 * 80 + vec * 16, 16)] = a1
    out_v[pl.ds(2 * 80 + vec * 16, 16)] = a2

  for c in range(_C):
    pltpu.sync_copy(out_v.at[pl.ds(c * 80, _PPW)],
                    out_hbm.at[pl.ds(c * _NPIX + base, _PPW)])


@jax.jit
def _dsepconv_sc(inp_flat, vt, ht, ox, oy, mk):
  mesh = plsc.VectorSubcoreMesh(core_axis_name="c", subcore_axis_name="s")
  run = pl.kernel(
      _dsep_body,
      out_type=jax.ShapeDtypeStruct((_C * _NPIX,), jnp.float32),
      mesh=mesh,
      compiler_params=pltpu.CompilerParams(
          needs_layout_passes=False, use_tc_tiling_on_sc=False),
      scratch_types=[
          pltpu.VMEM((_C * _HI * _WI,), jnp.float32),
          pltpu.VMEM((_F, _PPW), jnp.float32),
          pltpu.VMEM((_F, _PPW), jnp.float32),
          pltpu.VMEM((_K, _PPW), jnp.float32),
          pltpu.VMEM((_K, _PPW), jnp.float32),
          pltpu.VMEM((_K, _PPW), jnp.float32),
          pltpu.VMEM((_C * 80,), jnp.float32),
          pltpu.SemaphoreType.DMA,
      ],
  )
  return run(inp_flat, vt, ht, ox, oy, mk)


def kernel(tensorInput, tensorVertical, tensorHorizontal, tensorOffsetX,
           tensorOffsetY, tensorMask):
  inp_flat = tensorInput.reshape(_C * _HI * _WI)
  vt = tensorVertical.reshape(_F, _NPIX)
  ht = tensorHorizontal.reshape(_F, _NPIX)
  ox = tensorOffsetX.reshape(_K, _NPIX)
  oy = tensorOffsetY.reshape(_K, _NPIX)
  mk = tensorMask.reshape(_K, _NPIX)
  out = _dsepconv_sc(inp_flat, vt, ht, ox, oy, mk)
  return out.reshape(1, _C, _HO, _WO)
